# baseline (speedup 1.0000x reference)
import functools

import jax
import jax.numpy as jnp
import numpy as np
from jax import lax
from jax.experimental import pallas as pl
from jax.experimental.pallas import tpu as pltpu
from jax.experimental.pallas import tpu_sc as plsc

N = 10000
E = 320000
D = 128
H = 16
EF = 4
NG = 20
RF = EF * NG
R_MAX = 10.0
DH = D // H

NC = 2
NS = 16
NW = NC * NS
CH = 80
NCH = E // CH

BE = 512
GE = E // BE
BN = 1000
GN = N // BN

_INV_SQRT_DH = 1.0 / np.sqrt(DH)

_E1 = np.zeros((EF, 128), np.float32)
_R1 = np.zeros((EF, 128), np.float32)
_R2 = np.zeros((32, 128), np.float32)
for _f in range(EF):
    _E1[_f, _f] = 1.0
    for _g in range(NG):
        _R1[_f, 4 + _f * NG + _g] = 1.0
        _R2[_g, 4 + _f * NG + _g] = 1.0
_S = np.zeros((128, 16), np.float32)
_B2 = np.zeros((16, 128), np.float32)
for _h in range(H):
    for _d in range(DH):
        _S[_h * DH + _d, _h] = 1.0
        _B2[_h, _h * DH + _d] = 1.0
_OFF = np.linspace(0.0, R_MAX, NG).astype(np.float32)
_OFFC = np.zeros((32, 1), np.float32)
_OFFC[:NG, 0] = _OFF
_COEFF = float(-0.5 / (_OFF[1] - _OFF[0]) ** 2)


def _ln_relu(hh, g, be):
    mu = jnp.mean(hh, axis=-1, keepdims=True)
    var = jnp.mean((hh - mu) ** 2, axis=-1, keepdims=True)
    return jnp.maximum((hh - mu) * lax.rsqrt(var + 1e-5) * g + be, 0.0)


def _q_body(h_ref, w1, b1, g, be, w2, b2, o_ref):
    hh = jnp.dot(h_ref[...], w1[...], preferred_element_type=jnp.float32) + b1[...]
    a = _ln_relu(hh, g[...], be[...])
    o_ref[...] = jnp.dot(a, w2[...], preferred_element_type=jnp.float32) + b2[...]


def _q_mlp(h, p):
    full = lambda s: pl.BlockSpec(s, lambda i: (0,) * len(s))
    return pl.pallas_call(
        _q_body,
        grid=(GN,),
        in_specs=[pl.BlockSpec((BN, D), lambda i: (i, 0)),
                  full((D, D)), full((1, D)), full((1, D)), full((1, D)),
                  full((D, D)), full((1, D))],
        out_specs=pl.BlockSpec((BN, D), lambda i: (i, 0)),
        out_shape=jax.ShapeDtypeStruct((N, D), jnp.float32),
    )(h, p['W1'], p['b1'][None], p['g'][None], p['be'][None], p['W2'], p['b2'][None])


def _gather_body(ht, qt, x0, x1, x2, src, dst, oS, oD, oQ, oR,
                 x0v, x1v, x2v, sidx, didx, bS, bD, bQ, d2b,
                 semi, semg, sems):
    c = lax.axis_index("c")
    s = lax.axis_index("s")
    wid = s * NC + c
    nt = NCH // NW + jnp.where(wid < NCH % NW, 1, 0)
    pltpu.sync_copy(x0, x0v)
    pltpu.sync_copy(x1, x1v)
    pltpu.sync_copy(x2, x2v)

    def _drain_stores():
        pltpu.make_async_copy(bS, oS.at[pl.ds(0, CH)], sems).wait()
        pltpu.make_async_copy(bD, oD.at[pl.ds(0, CH)], sems).wait()
        pltpu.make_async_copy(bQ, oQ.at[pl.ds(0, CH)], sems).wait()
        pltpu.make_async_copy(d2b, oR.at[pl.ds(0, CH)], sems).wait()

    def body(i, _):
        off = pl.multiple_of((wid + i * NW) * CH, 8)
        l1 = pltpu.async_copy(src.at[pl.ds(off, CH)], sidx, semi)
        l2 = pltpu.async_copy(dst.at[pl.ds(off, CH)], didx, semi)

        @pl.when(i > 0)
        def _():
            _drain_stores()

        l1.wait()
        l2.wait()
        g1 = pltpu.async_copy(ht.at[sidx], bS, semg)
        g2 = pltpu.async_copy(ht.at[didx], bD, semg)
        g3 = pltpu.async_copy(qt.at[didx], bQ, semg)
        for j in range(CH // 16):
            si = sidx[pl.ds(j * 16, 16)]
            di = didx[pl.ds(j * 16, 16)]
            acc = jnp.zeros((16,), jnp.float32)
            for colv in (x0v, x1v, x2v):
                r = plsc.load_gather(colv, [di]) - plsc.load_gather(colv, [si])
                acc = acc + r * r
            d2b[pl.ds(j * 16, 16)] = acc
        g1.wait()
        g2.wait()
        g3.wait()
        pltpu.async_copy(bS, oS.at[pl.ds(off, CH)], sems)
        pltpu.async_copy(bD, oD.at[pl.ds(off, CH)], sems)
        pltpu.async_copy(bQ, oQ.at[pl.ds(off, CH)], sems)
        pltpu.async_copy(d2b, oR.at[pl.ds(off, CH)], sems)
        return 0

    lax.fori_loop(0, nt, body, 0)
    _drain_stores()


@functools.cache
def _gather():
    return pl.kernel(
        _gather_body,
        out_type=(jax.ShapeDtypeStruct((E, D), jnp.float32),
                  jax.ShapeDtypeStruct((E, D), jnp.float32),
                  jax.ShapeDtypeStruct((E, D), jnp.float32),
                  jax.ShapeDtypeStruct((E,), jnp.float32)),
        mesh=plsc.VectorSubcoreMesh(core_axis_name="c", subcore_axis_name="s",
                                    num_cores=NC, num_subcores=NS),
        compiler_params=pltpu.CompilerParams(needs_layout_passes=False),
        scratch_types=[pltpu.VMEM((N,), jnp.float32), pltpu.VMEM((N,), jnp.float32),
                       pltpu.VMEM((N,), jnp.float32),
                       pltpu.VMEM((CH,), jnp.int32), pltpu.VMEM((CH,), jnp.int32),
                       pltpu.VMEM((CH, D), jnp.float32),
                       pltpu.VMEM((CH, D), jnp.float32),
                       pltpu.VMEM((CH, D), jnp.float32),
                       pltpu.VMEM((CH,), jnp.float32),
                       pltpu.SemaphoreType.DMA, pltpu.SemaphoreType.DMA,
                       pltpu.SemaphoreType.DMA],
    )


def _edge_body(hs_ref, hd_ref, q_ref, d2_ref, ea_ref,
               e1, r1, r2, offc, ewwp, ewb,
               wk1, bk1, gk, bek, wk2, bk2,
               wv1, bv1, gv, bev, wv2, bv2,
               sel, b2, omv_ref, op_ref):
    hs = hs_ref[...]
    hd = hd_ref[...]
    qd = q_ref[...]
    eap = ea_ref[...]

    dist_t = jnp.sqrt(d2_ref[0])
    dfp_t = jnp.exp(_COEFF * (dist_t - offc[...]) ** 2)
    rfB = lax.dot_general(dfp_t, r2[...], (((0,), (0,)), ((), ())),
                          preferred_element_type=jnp.float32)
    kvA = (jnp.dot(eap, e1[...], preferred_element_type=jnp.float32)
           + jnp.dot(eap, r1[...], preferred_element_type=jnp.float32) * rfB)
    kvcat = jnp.concatenate([kvA, hd, hs], axis=1)

    hhk = jnp.dot(kvcat, wk1[...], preferred_element_type=jnp.float32) + bk1[...]
    ak = _ln_relu(hhk, gk[...], bek[...])
    k = jnp.dot(ak, wk2[...], preferred_element_type=jnp.float32) + bk2[...]

    hhv = jnp.dot(kvcat, wv1[...], preferred_element_type=jnp.float32) + bv1[...]
    av = _ln_relu(hhv, gv[...], bev[...])
    v = jnp.dot(av, wv2[...], preferred_element_type=jnp.float32) + bv2[...]

    logit = jnp.sum(kvA * ewwp[...], axis=1, keepdims=True) + ewb[...]
    vw = v * jax.nn.sigmoid(logit)

    p = jnp.exp(jnp.dot(qd * k, sel[...], preferred_element_type=jnp.float32)
                * _INV_SQRT_DH)
    p128 = jnp.dot(p, b2[...], preferred_element_type=jnp.float32)
    omv_ref[...] = p128 * vw
    op_ref[...] = p128


def _edge_compute(HS, HD, Q, D2, ea, consts):
    full = lambda s: pl.BlockSpec(s, lambda i: (0,) * len(s))
    in_specs = [pl.BlockSpec((BE, D), lambda i: (i, 0)),
                pl.BlockSpec((BE, D), lambda i: (i, 0)),
                pl.BlockSpec((BE, D), lambda i: (i, 0)),
                pl.BlockSpec((1, 1, BE), lambda i: (i, 0, 0)),
                pl.BlockSpec((BE, EF), lambda i: (i, 0))]
    in_specs += [full(c.shape) for c in consts]
    return pl.pallas_call(
        _edge_body,
        grid=(GE,),
        in_specs=in_specs,
        out_specs=[pl.BlockSpec((BE, D), lambda i: (i, 0)),
                   pl.BlockSpec((BE, D), lambda i: (i, 0))],
        out_shape=[jax.ShapeDtypeStruct((E, D), jnp.float32),
                   jax.ShapeDtypeStruct((E, D), jnp.float32)],
    )(HS, HD, Q, D2, ea, *consts)


_ROWS = 624
_TAIL0 = NS * _ROWS
_TAIL = N - _TAIL0


def _make_scatter_body(W):
    def body_fn(dst, pmx, z, o, idxv, pmv, tbl, seml):
        c = lax.axis_index("c")
        s = lax.axis_index("s")
        wid = s * NC + c
        nt = NCH // NW + jnp.where(wid < NCH % NW, 1, 0)
        r0 = s * _ROWS
        pltpu.sync_copy(z.at[pl.ds(r0, _ROWS)], tbl.at[pl.ds(r0, _ROWS)])

        @pl.when(s == 0)
        def _():
            pltpu.sync_copy(z.at[pl.ds(_TAIL0, _TAIL)],
                            tbl.at[pl.ds(_TAIL0, _TAIL)])

        plsc.subcore_barrier()

        def body(i, _):
            off = pl.multiple_of((wid + i * NW) * CH, 8)
            l1 = pltpu.async_copy(dst.at[pl.ds(off, CH)], idxv, seml)
            l2 = pltpu.async_copy(pmx.at[pl.ds(off, CH)], pmv, seml)
            l1.wait()
            l2.wait()
            pltpu.sync_copy(pmv, tbl.at[idxv], add=True)
            return 0

        lax.fori_loop(0, nt, body, 0)
        plsc.subcore_barrier()
        pltpu.sync_copy(tbl.at[pl.ds(r0, _ROWS)], o.at[c, pl.ds(r0, _ROWS)])

        @pl.when(s == 0)
        def _():
            pltpu.sync_copy(tbl.at[pl.ds(_TAIL0, _TAIL)],
                            o.at[c, pl.ds(_TAIL0, _TAIL)])

    return body_fn


@functools.cache
def _scatter(W, tiled):
    cp = None if tiled else pltpu.CompilerParams(use_tc_tiling_on_sc=False)
    return pl.kernel(
        _make_scatter_body(W),
        out_type=jax.ShapeDtypeStruct((NC, N, W), jnp.float32),
        mesh=plsc.VectorSubcoreMesh(core_axis_name="c", subcore_axis_name="s",
                                    num_cores=NC, num_subcores=NS),
        compiler_params=cp,
        scratch_types=[pltpu.VMEM((CH,), jnp.int32),
                       pltpu.VMEM((CH, W), jnp.float32),
                       pltpu.VMEM_SHARED((N, W), jnp.float32),
                       pltpu.SemaphoreType.DMA],
    )


def _final_body(tmv_ref, tp_ref, h_ref, w1a, w1b, b1, g, be, w2, b2b, o_ref):
    s2 = tmv_ref[0] + tmv_ref[1]
    den = tp_ref[0] + tp_ref[1]
    agg = s2 / (den + 1e-16)
    hb = h_ref[...]
    hh = (jnp.dot(agg, w1a[...], preferred_element_type=jnp.float32)
          + jnp.dot(hb, w1b[...], preferred_element_type=jnp.float32) + b1[...])
    a = _ln_relu(hh, g[...], be[...])
    o_ref[...] = jnp.dot(a, w2[...], preferred_element_type=jnp.float32) + b2b[...] + hb


def _final(parts_mv, parts_p, h, p):
    full = lambda s: pl.BlockSpec(s, lambda i: (0,) * len(s))
    return pl.pallas_call(
        _final_body,
        grid=(GN,),
        in_specs=[pl.BlockSpec((NC, BN, D), lambda i: (0, i, 0)),
                  pl.BlockSpec((NC, BN, D), lambda i: (0, i, 0)),
                  pl.BlockSpec((BN, D), lambda i: (i, 0)),
                  full((D, D)), full((D, D)), full((1, D)),
                  full((1, D)), full((1, D)), full((D, D)), full((1, D))],
        out_specs=pl.BlockSpec((BN, D), lambda i: (i, 0)),
        out_shape=jax.ShapeDtypeStruct((N, D), jnp.float32),
    )(parts_mv, parts_p, h, p['W1'][:D], p['W1'][D:],
      p['b1'][None], p['g'][None], p['be'][None], p['W2'], p['b2'][None])


def kernel(x, h, edge_attr, edge_index, e_w, hk, hv, hq, ew_W, ew_b, nout):
    del e_w
    src = edge_index[0]
    dst = edge_index[1]

    q = _q_mlp(h, hq)
    HS, HD, Q, D2 = _gather()(h, q, x[:, 0], x[:, 1], x[:, 2], src, dst)

    def mk_w1(p):
        w = p['W1']
        return jnp.concatenate(
            [w[0:84], jnp.zeros((44, D), jnp.float32), w[84:212], w[212:340]], axis=0)

    ewwp = jnp.zeros((1, 128), jnp.float32).at[0, 4:84].set(ew_W[:, 0])
    consts = (jnp.asarray(_E1), jnp.asarray(_R1), jnp.asarray(_R2),
              jnp.asarray(_OFFC), ewwp, ew_b[None],
              mk_w1(hk), hk['b1'][None], hk['g'][None], hk['be'][None],
              hk['W2'], hk['b2'][None],
              mk_w1(hv), hv['b1'][None], hv['g'][None], hv['be'][None],
              hv['W2'], hv['b2'][None],
              jnp.asarray(_S), jnp.asarray(_B2))
    mv, pp = _edge_compute(HS, HD, Q, D2.reshape(GE, 1, BE), edge_attr, consts)

    zn = jnp.zeros((N, D), jnp.float32)
    parts_mv = _scatter(D, True)(dst, mv, zn)
    parts_p = _scatter(D, True)(dst, pp, zn)
    return _final(parts_mv, parts_p, h, nout)


# speedup vs baseline: 1.2843x; 1.2843x over previous
import functools

import jax
import jax.numpy as jnp
import numpy as np
from jax import lax
from jax.experimental import pallas as pl
from jax.experimental.pallas import tpu as pltpu
from jax.experimental.pallas import tpu_sc as plsc

N = 10000
E = 320000
D = 128
H = 16
EF = 4
NG = 20
RF = EF * NG
R_MAX = 10.0
DH = D // H

NC = 2
NS = 16
NW = NC * NS
CH = 80

K = 5
EC = E // K
NCH = EC // CH
ZT = NCH // NW

BE = 512
GE = EC // BE
BN = 1000
GN = N // BN

_INV_SQRT_DH = 1.0 / np.sqrt(DH)

_E1 = np.zeros((EF, 128), np.float32)
_R1 = np.zeros((EF, 128), np.float32)
_R2 = np.zeros((32, 128), np.float32)
for _f in range(EF):
    _E1[_f, _f] = 1.0
    for _g in range(NG):
        _R1[_f, 4 + _f * NG + _g] = 1.0
        _R2[_g, 4 + _f * NG + _g] = 1.0
_S = np.zeros((128, 16), np.float32)
_B2 = np.zeros((16, 128), np.float32)
for _h in range(H):
    for _d in range(DH):
        _S[_h * DH + _d, _h] = 1.0
        _B2[_h, _h * DH + _d] = 1.0
_OFF = np.linspace(0.0, R_MAX, NG).astype(np.float32)
_OFFC = np.zeros((32, 1), np.float32)
_OFFC[:NG, 0] = _OFF
_COEFF = float(-0.5 / (_OFF[1] - _OFF[0]) ** 2)


def _ln_relu(hh, g, be):
    mu = jnp.mean(hh, axis=-1, keepdims=True)
    var = jnp.mean((hh - mu) ** 2, axis=-1, keepdims=True)
    return jnp.maximum((hh - mu) * lax.rsqrt(var + 1e-5) * g + be, 0.0)


def _q_body(h_ref, w1, b1, g, be, w2, b2, o_ref):
    hh = jnp.dot(h_ref[...], w1[...], preferred_element_type=jnp.float32) + b1[...]
    a = _ln_relu(hh, g[...], be[...])
    o_ref[...] = jnp.dot(a, w2[...], preferred_element_type=jnp.float32) + b2[...]


def _q_mlp(h, p):
    full = lambda s: pl.BlockSpec(s, lambda i: (0,) * len(s))
    return pl.pallas_call(
        _q_body,
        grid=(GN,),
        in_specs=[pl.BlockSpec((BN, D), lambda i: (i, 0)),
                  full((D, D)), full((1, D)), full((1, D)), full((1, D)),
                  full((D, D)), full((1, D))],
        out_specs=pl.BlockSpec((BN, D), lambda i: (i, 0)),
        out_shape=jax.ShapeDtypeStruct((N, D), jnp.float32),
    )(h, p['W1'], p['b1'][None], p['g'][None], p['be'][None], p['W2'], p['b2'][None])


def _gather_body(ht, tq, x0, x1, x2, src, dst, oS, oDQ, oR,
                 x0v, x1v, x2v,
                 sidx0, didx0, bS0, bDQ0, d2b0,
                 sidx1, didx1, bS1, bDQ1, d2b1,
                 seml0, seml1, semg0, semg1, sems0, sems1):
    c = lax.axis_index("c")
    s = lax.axis_index("s")
    wid = s * NC + c
    pltpu.sync_copy(x0, x0v)
    pltpu.sync_copy(x1, x1v)
    pltpu.sync_copy(x2, x2v)

    bufs = ((sidx0, didx0, bS0, bDQ0, d2b0, seml0, semg0, sems0),
            (sidx1, didx1, bS1, bDQ1, d2b1, seml1, semg1, sems1))

    def off_of(i):
        return pl.multiple_of((wid + i * NW) * CH, 8)

    def load_idx(i, P):
        sidx, didx, _, _, _, seml, _, _ = bufs[P]
        pltpu.async_copy(src.at[pl.ds(off_of(i), CH)], sidx, seml)
        pltpu.async_copy(dst.at[pl.ds(off_of(i), CH)], didx, seml)

    def wait_idx(P):
        sidx, didx, _, _, _, seml, _, _ = bufs[P]
        pltpu.make_async_copy(src.at[pl.ds(0, CH)], sidx, seml).wait()
        pltpu.make_async_copy(dst.at[pl.ds(0, CH)], didx, seml).wait()

    def drain_stores(P):
        _, _, bS, bDQ, d2b, _, _, sems = bufs[P]
        pltpu.make_async_copy(bS, oS.at[pl.ds(0, CH)], sems).wait()
        pltpu.make_async_copy(bDQ, oDQ.at[pl.ds(0, CH)], sems).wait()
        pltpu.make_async_copy(d2b, oR.at[pl.ds(0, CH)], sems).wait()

    def process(i, P, load_next, drain_first):
        sidx, didx, bS, bDQ, d2b, seml, semg, sems = bufs[P]
        wait_idx(P)
        if load_next:
            load_idx(i + 1, 1 - P)
        if drain_first is not None:
            @pl.when(drain_first)
            def _():
                drain_stores(P)
        else:
            drain_stores(P)
        g1 = pltpu.async_copy(ht.at[sidx], bS, semg)
        g2 = pltpu.async_copy(tq.at[didx], bDQ, semg)
        for j in range(CH // 16):
            si = sidx[pl.ds(j * 16, 16)]
            di = didx[pl.ds(j * 16, 16)]
            acc = jnp.zeros((16,), jnp.float32)
            for colv in (x0v, x1v, x2v):
                r = plsc.load_gather(colv, [di]) - plsc.load_gather(colv, [si])
                acc = acc + r * r
            d2b[pl.ds(j * 16, 16)] = acc
        g1.wait()
        g2.wait()
        off = off_of(i)
        pltpu.async_copy(bS, oS.at[pl.ds(off, CH)], sems)
        pltpu.async_copy(bDQ, oDQ.at[pl.ds(off, CH)], sems)
        pltpu.async_copy(d2b, oR.at[pl.ds(off, CH)], sems)

    load_idx(0, 0)

    def body(j, _):
        process(2 * j, 0, True, j > 0)
        process(2 * j + 1, 1, True, j > 0)
        return 0

    lax.fori_loop(0, (ZT - 1) // 2, body, 0)
    process(ZT - 1, 0, False, None)
    drain_stores(1)
    drain_stores(0)


@functools.cache
def _gather():
    return pl.kernel(
        _gather_body,
        out_type=(jax.ShapeDtypeStruct((EC, D), jnp.float32),
                  jax.ShapeDtypeStruct((EC, 2 * D), jnp.float32),
                  jax.ShapeDtypeStruct((EC,), jnp.float32)),
        mesh=plsc.VectorSubcoreMesh(core_axis_name="c", subcore_axis_name="s",
                                    num_cores=NC, num_subcores=NS),
        compiler_params=pltpu.CompilerParams(needs_layout_passes=False),
        scratch_types=[pltpu.VMEM((N,), jnp.float32), pltpu.VMEM((N,), jnp.float32),
                       pltpu.VMEM((N,), jnp.float32)]
                      + 2 * [pltpu.VMEM((CH,), jnp.int32),
                             pltpu.VMEM((CH,), jnp.int32),
                             pltpu.VMEM((CH, D), jnp.float32),
                             pltpu.VMEM((CH, 2 * D), jnp.float32),
                             pltpu.VMEM((CH,), jnp.float32)]
                      + 6 * [pltpu.SemaphoreType.DMA],
    )


def _edge_body(hs_ref, dq_ref, d2_ref, ea_ref,
               e1, r1, r2, offc, ewwp, ewb,
               wk1, bk1, gk, bek, wk2, bk2,
               wv1, bv1, gv, bev, wv2, bv2,
               sel, b2, omv_ref, op_ref):
    hs = hs_ref[...]
    hd = dq_ref[:, :D]
    qd = dq_ref[:, D:]
    eap = ea_ref[...]

    dist_t = jnp.sqrt(d2_ref[0])
    dfp_t = jnp.exp(_COEFF * (dist_t - offc[...]) ** 2)
    rfB = lax.dot_general(dfp_t, r2[...], (((0,), (0,)), ((), ())),
                          preferred_element_type=jnp.float32)
    kvA = (jnp.dot(eap, e1[...], preferred_element_type=jnp.float32)
           + jnp.dot(eap, r1[...], preferred_element_type=jnp.float32) * rfB)
    kvcat = jnp.concatenate([kvA, hd, hs], axis=1)

    hhk = jnp.dot(kvcat, wk1[...], preferred_element_type=jnp.float32) + bk1[...]
    ak = _ln_relu(hhk, gk[...], bek[...])
    k = jnp.dot(ak, wk2[...], preferred_element_type=jnp.float32) + bk2[...]

    hhv = jnp.dot(kvcat, wv1[...], preferred_element_type=jnp.float32) + bv1[...]
    av = _ln_relu(hhv, gv[...], bev[...])
    v = jnp.dot(av, wv2[...], preferred_element_type=jnp.float32) + bv2[...]

    logit = jnp.sum(kvA * ewwp[...], axis=1, keepdims=True) + ewb[...]
    vw = v * jax.nn.sigmoid(logit)

    p = jnp.exp(jnp.dot(qd * k, sel[...], preferred_element_type=jnp.float32)
                * _INV_SQRT_DH)
    p128 = jnp.dot(p, b2[...], preferred_element_type=jnp.float32)
    omv_ref[...] = p128 * vw
    op_ref[...] = p128


def _edge_compute(HS, DQ, D2, ea, consts):
    full = lambda s: pl.BlockSpec(s, lambda i: (0,) * len(s))
    in_specs = [pl.BlockSpec((BE, D), lambda i: (i, 0)),
                pl.BlockSpec((BE, 2 * D), lambda i: (i, 0)),
                pl.BlockSpec((1, 1, BE), lambda i: (i, 0, 0)),
                pl.BlockSpec((BE, EF), lambda i: (i, 0))]
    in_specs += [full(c.shape) for c in consts]
    return pl.pallas_call(
        _edge_body,
        grid=(GE,),
        in_specs=in_specs,
        out_specs=[pl.BlockSpec((BE, D), lambda i: (i, 0)),
                   pl.BlockSpec((BE, D), lambda i: (i, 0))],
        out_shape=[jax.ShapeDtypeStruct((EC, D), jnp.float32),
                   jax.ShapeDtypeStruct((EC, D), jnp.float32)],
    )(HS, DQ, D2, ea, *consts)


_ROWS = 624
_TAIL0 = NS * _ROWS
_TAIL = N - _TAIL0


def _scatter_body(dst, pmx, z, o, idxv0, pmv0, idxv1, pmv1, tbl,
                  seml0, seml1, semx0, semx1):
    c = lax.axis_index("c")
    s = lax.axis_index("s")
    wid = s * NC + c
    r0 = s * _ROWS
    pltpu.sync_copy(z.at[pl.ds(r0, _ROWS)], tbl.at[pl.ds(r0, _ROWS)])

    @pl.when(s == 0)
    def _():
        pltpu.sync_copy(z.at[pl.ds(_TAIL0, _TAIL)], tbl.at[pl.ds(_TAIL0, _TAIL)])

    plsc.subcore_barrier()
    bufs = ((idxv0, pmv0, seml0, semx0), (idxv1, pmv1, seml1, semx1))

    def off_of(i):
        return pl.multiple_of((wid + i * NW) * CH, 8)

    def load(i, P):
        idxv, pmv, seml, _ = bufs[P]
        pltpu.async_copy(dst.at[pl.ds(off_of(i), CH)], idxv, seml)
        pltpu.async_copy(pmx.at[pl.ds(off_of(i), CH)], pmv, seml)

    def process(i, P, load_next, drain_other):
        idxv, pmv, seml, semx = bufs[P]
        pltpu.make_async_copy(dst.at[pl.ds(0, CH)], idxv, seml).wait()
        pltpu.make_async_copy(pmx.at[pl.ds(0, CH)], pmv, seml).wait()
        if drain_other is not None:
            oidxv, opmv, _, osemx = bufs[1 - P]

            @pl.when(drain_other)
            def _():
                pltpu.make_async_copy(opmv, tbl.at[oidxv], osemx).wait()
        if load_next:
            load(i + 1, 1 - P)
        pltpu.async_copy(pmv, tbl.at[idxv], semx, add=True)

    load(0, 0)

    def body(j, _):
        process(2 * j, 0, True, j > 0)
        process(2 * j + 1, 1, True, True)
        return 0

    lax.fori_loop(0, (ZT - 1) // 2, body, 0)
    process(ZT - 1, 0, False, True)
    pltpu.make_async_copy(pmv0, tbl.at[idxv0], semx0).wait()
    plsc.subcore_barrier()
    pltpu.sync_copy(tbl.at[pl.ds(r0, _ROWS)], o.at[c, pl.ds(r0, _ROWS)])

    @pl.when(s == 0)
    def _():
        pltpu.sync_copy(tbl.at[pl.ds(_TAIL0, _TAIL)], o.at[c, pl.ds(_TAIL0, _TAIL)])


@functools.cache
def _scatter():
    return pl.kernel(
        _scatter_body,
        out_type=jax.ShapeDtypeStruct((NC, N, D), jnp.float32),
        mesh=plsc.VectorSubcoreMesh(core_axis_name="c", subcore_axis_name="s",
                                    num_cores=NC, num_subcores=NS),
        scratch_types=2 * [pltpu.VMEM((CH,), jnp.int32),
                           pltpu.VMEM((CH, D), jnp.float32)]
                      + [pltpu.VMEM_SHARED((N, D), jnp.float32)]
                      + 4 * [pltpu.SemaphoreType.DMA],
    )


def _final_body(*refs):
    (m0, m1, m2, m3, m4, p0, p1, p2, p3, p4, h_ref,
     w1a, w1b, b1, g, be, w2, b2b, o_ref) = refs
    s2 = sum(t[0] + t[1] for t in (m0, m1, m2, m3, m4))
    den = sum(t[0] + t[1] for t in (p0, p1, p2, p3, p4))
    agg = s2 / (den + 1e-16)
    hb = h_ref[...]
    hh = (jnp.dot(agg, w1a[...], preferred_element_type=jnp.float32)
          + jnp.dot(hb, w1b[...], preferred_element_type=jnp.float32) + b1[...])
    a = _ln_relu(hh, g[...], be[...])
    o_ref[...] = jnp.dot(a, w2[...], preferred_element_type=jnp.float32) + b2b[...] + hb


def _final(parts_mv, parts_p, h, p):
    full = lambda s: pl.BlockSpec(s, lambda i: (0,) * len(s))
    part_spec = pl.BlockSpec((NC, BN, D), lambda i: (0, i, 0))
    return pl.pallas_call(
        _final_body,
        grid=(GN,),
        in_specs=2 * K * [part_spec]
                 + [pl.BlockSpec((BN, D), lambda i: (i, 0)),
                    full((D, D)), full((D, D)), full((1, D)),
                    full((1, D)), full((1, D)), full((D, D)), full((1, D))],
        out_specs=pl.BlockSpec((BN, D), lambda i: (i, 0)),
        out_shape=jax.ShapeDtypeStruct((N, D), jnp.float32),
    )(*parts_mv, *parts_p, h, p['W1'][:D], p['W1'][D:],
      p['b1'][None], p['g'][None], p['be'][None], p['W2'], p['b2'][None])


def kernel(x, h, edge_attr, edge_index, e_w, hk, hv, hq, ew_W, ew_b, nout):
    del e_w
    src = edge_index[0]
    dst = edge_index[1]

    q = _q_mlp(h, hq)
    tq = jnp.concatenate([h, q], axis=1)

    def mk_w1(p):
        w = p['W1']
        return jnp.concatenate(
            [w[0:84], jnp.zeros((44, D), jnp.float32), w[84:212], w[212:340]], axis=0)

    ewwp = jnp.zeros((1, 128), jnp.float32).at[0, 4:84].set(ew_W[:, 0])
    consts = (jnp.asarray(_E1), jnp.asarray(_R1), jnp.asarray(_R2),
              jnp.asarray(_OFFC), ewwp, ew_b[None],
              mk_w1(hk), hk['b1'][None], hk['g'][None], hk['be'][None],
              hk['W2'], hk['b2'][None],
              mk_w1(hv), hv['b1'][None], hv['g'][None], hv['be'][None],
              hv['W2'], hv['b2'][None],
              jnp.asarray(_S), jnp.asarray(_B2))
    zn = jnp.zeros((N, D), jnp.float32)
    parts_mv, parts_p = [], []
    for k in range(K):
        sl = slice(k * EC, (k + 1) * EC)
        srck, dstk = src[sl], dst[sl]
        HSk, DQk, D2k = _gather()(h, tq, x[:, 0], x[:, 1], x[:, 2], srck, dstk)
        mvk, ppk = _edge_compute(HSk, DQk, D2k.reshape(GE, 1, BE),
                                 edge_attr[sl], consts)
        parts_mv.append(_scatter()(dstk, mvk, zn))
        parts_p.append(_scatter()(dstk, ppk, zn))
    return _final(parts_mv, parts_p, h, nout)


# speedup vs baseline: 1.3203x; 1.0281x over previous
import functools

import jax
import jax.numpy as jnp
import numpy as np
from jax import lax
from jax.experimental import pallas as pl
from jax.experimental.pallas import tpu as pltpu
from jax.experimental.pallas import tpu_sc as plsc

N = 10000
E = 320000
D = 128
H = 16
EF = 4
NG = 20
RF = EF * NG
R_MAX = 10.0
DH = D // H

NC = 2
NS = 16
NW = NC * NS
CH = 80

K = 5
EC = E // K
NCH = EC // CH
ZT = NCH // NW

BE = 512
GE = EC // BE
BN = 1000
GN = N // BN

_INV_SQRT_DH = 1.0 / np.sqrt(DH)

_E1 = np.zeros((EF, 128), np.float32)
_R1 = np.zeros((EF, 128), np.float32)
_R2 = np.zeros((32, 128), np.float32)
for _f in range(EF):
    _E1[_f, _f] = 1.0
    for _g in range(NG):
        _R1[_f, 4 + _f * NG + _g] = 1.0
        _R2[_g, 4 + _f * NG + _g] = 1.0
_S = np.zeros((128, 16), np.float32)
_B2 = np.zeros((16, 128), np.float32)
for _h in range(H):
    for _d in range(DH):
        _S[_h * DH + _d, _h] = 1.0
        _B2[_h, _h * DH + _d] = 1.0
_OFF = np.linspace(0.0, R_MAX, NG).astype(np.float32)
_OFFC = np.zeros((32, 1), np.float32)
_OFFC[:NG, 0] = _OFF
_COEFF = float(-0.5 / (_OFF[1] - _OFF[0]) ** 2)


def _ln_relu(hh, g, be, ones):
    del ones
    mu = jnp.mean(hh, axis=-1, keepdims=True)
    msq = jnp.mean(hh * hh, axis=-1, keepdims=True)
    var = msq - mu * mu
    return jnp.maximum((hh - mu) * lax.rsqrt(var + 1e-5) * g + be, 0.0)


def _q_body(h_ref, w1, b1, g, be, w2, b2, ones, o_ref):
    hh = jnp.dot(h_ref[...], w1[...], preferred_element_type=jnp.float32) + b1[...]
    a = _ln_relu(hh, g[...], be[...], ones[...])
    o_ref[...] = jnp.dot(a, w2[...], preferred_element_type=jnp.float32) + b2[...]


def _q_mlp(h, p):
    full = lambda s: pl.BlockSpec(s, lambda i: (0,) * len(s))
    return pl.pallas_call(
        _q_body,
        grid=(GN,),
        in_specs=[pl.BlockSpec((BN, D), lambda i: (i, 0)),
                  full((D, D)), full((1, D)), full((1, D)), full((1, D)),
                  full((D, D)), full((1, D)), full((D, 1))],
        out_specs=pl.BlockSpec((BN, D), lambda i: (i, 0)),
        out_shape=jax.ShapeDtypeStruct((N, D), jnp.float32),
    )(h, p['W1'], p['b1'][None], p['g'][None], p['be'][None], p['W2'],
      p['b2'][None], jnp.ones((D, 1), jnp.float32))


def _gather_body(ht, tq, x0, x1, x2, src, dst, oS, oDQ, oR,
                 x0v, x1v, x2v,
                 sidx0, didx0, bS0, bDQ0, d2b0,
                 sidx1, didx1, bS1, bDQ1, d2b1,
                 seml0, seml1, semg0, semg1, sems0, sems1):
    c = lax.axis_index("c")
    s = lax.axis_index("s")
    wid = s * NC + c
    pltpu.sync_copy(x0, x0v)
    pltpu.sync_copy(x1, x1v)
    pltpu.sync_copy(x2, x2v)

    bufs = ((sidx0, didx0, bS0, bDQ0, d2b0, seml0, semg0, sems0),
            (sidx1, didx1, bS1, bDQ1, d2b1, seml1, semg1, sems1))

    def off_of(i):
        return pl.multiple_of((wid + i * NW) * CH, 8)

    def load_idx(i, P):
        sidx, didx, _, _, _, seml, _, _ = bufs[P]
        pltpu.async_copy(src.at[pl.ds(off_of(i), CH)], sidx, seml)
        pltpu.async_copy(dst.at[pl.ds(off_of(i), CH)], didx, seml)

    def wait_idx(P):
        sidx, didx, _, _, _, seml, _, _ = bufs[P]
        pltpu.make_async_copy(src.at[pl.ds(0, CH)], sidx, seml).wait()
        pltpu.make_async_copy(dst.at[pl.ds(0, CH)], didx, seml).wait()

    def drain_stores(P):
        _, _, bS, bDQ, d2b, _, _, sems = bufs[P]
        pltpu.make_async_copy(bS, oS.at[pl.ds(0, CH)], sems).wait()
        pltpu.make_async_copy(bDQ, oDQ.at[pl.ds(0, CH)], sems).wait()
        pltpu.make_async_copy(d2b, oR.at[pl.ds(0, CH)], sems).wait()

    def process(i, P, load_next, drain_first):
        sidx, didx, bS, bDQ, d2b, seml, semg, sems = bufs[P]
        wait_idx(P)
        if load_next:
            load_idx(i + 1, 1 - P)
        if drain_first is not None:
            @pl.when(drain_first)
            def _():
                drain_stores(P)
        else:
            drain_stores(P)
        g1 = pltpu.async_copy(ht.at[sidx], bS, semg)
        g2 = pltpu.async_copy(tq.at[didx], bDQ, semg)
        for j in range(CH // 16):
            si = sidx[pl.ds(j * 16, 16)]
            di = didx[pl.ds(j * 16, 16)]
            acc = jnp.zeros((16,), jnp.float32)
            for colv in (x0v, x1v, x2v):
                r = plsc.load_gather(colv, [di]) - plsc.load_gather(colv, [si])
                acc = acc + r * r
            d2b[pl.ds(j * 16, 16)] = acc
        g1.wait()
        g2.wait()
        off = off_of(i)
        pltpu.async_copy(bS, oS.at[pl.ds(off, CH)], sems)
        pltpu.async_copy(bDQ, oDQ.at[pl.ds(off, CH)], sems)
        pltpu.async_copy(d2b, oR.at[pl.ds(off, CH)], sems)

    load_idx(0, 0)

    def body(j, _):
        process(2 * j, 0, True, j > 0)
        process(2 * j + 1, 1, True, j > 0)
        return 0

    lax.fori_loop(0, (ZT - 1) // 2, body, 0)
    process(ZT - 1, 0, False, None)
    drain_stores(1)
    drain_stores(0)


@functools.cache
def _gather():
    return pl.kernel(
        _gather_body,
        out_type=(jax.ShapeDtypeStruct((EC, D), jnp.float32),
                  jax.ShapeDtypeStruct((EC, 2 * D), jnp.float32),
                  jax.ShapeDtypeStruct((EC,), jnp.float32)),
        mesh=plsc.VectorSubcoreMesh(core_axis_name="c", subcore_axis_name="s",
                                    num_cores=NC, num_subcores=NS),
        compiler_params=pltpu.CompilerParams(needs_layout_passes=False),
        scratch_types=[pltpu.VMEM((N,), jnp.float32), pltpu.VMEM((N,), jnp.float32),
                       pltpu.VMEM((N,), jnp.float32)]
                      + 2 * [pltpu.VMEM((CH,), jnp.int32),
                             pltpu.VMEM((CH,), jnp.int32),
                             pltpu.VMEM((CH, D), jnp.float32),
                             pltpu.VMEM((CH, 2 * D), jnp.float32),
                             pltpu.VMEM((CH,), jnp.float32)]
                      + 6 * [pltpu.SemaphoreType.DMA],
    )


def _edge_body(hs_ref, dq_ref, d2_ref, ea_ref,
               e1, r1, r2, offc, ewc, ewb,
               wk1, bk1, gk, bek, wk2, bk2,
               wv1, bv1, gv, bev, wv2, bv2,
               sel, b2, ones, omv_ref, op_ref):
    hs = hs_ref[...]
    hd = dq_ref[:, :D]
    qd = dq_ref[:, D:]
    eap = ea_ref[...]

    dist_t = jnp.sqrt(d2_ref[0])
    dfp_t = jnp.exp(_COEFF * (dist_t - offc[...]) ** 2)
    rfB = lax.dot_general(dfp_t, r2[...], (((0,), (0,)), ((), ())),
                          preferred_element_type=jnp.float32)
    kvA = (jnp.dot(eap, e1[...], preferred_element_type=jnp.float32)
           + jnp.dot(eap, r1[...], preferred_element_type=jnp.float32) * rfB)
    kvcat = jnp.concatenate([kvA, hd, hs], axis=1)

    hhk = jnp.dot(kvcat, wk1[...], preferred_element_type=jnp.float32) + bk1[...]
    ak = _ln_relu(hhk, gk[...], bek[...], ones[...])
    k = jnp.dot(ak, wk2[...], preferred_element_type=jnp.float32) + bk2[...]

    hhv = jnp.dot(kvcat, wv1[...], preferred_element_type=jnp.float32) + bv1[...]
    av = _ln_relu(hhv, gv[...], bev[...], ones[...])
    v = jnp.dot(av, wv2[...], preferred_element_type=jnp.float32) + bv2[...]

    logit = jnp.sum(kvA * ewc[...], axis=1, keepdims=True) + ewb[...]
    vw = v * jax.nn.sigmoid(logit)

    p = jnp.exp(jnp.dot(qd * k, sel[...], preferred_element_type=jnp.float32)
                * _INV_SQRT_DH)
    p128 = jnp.dot(p, b2[...], preferred_element_type=jnp.float32)
    omv_ref[...] = p128 * vw
    op_ref[...] = p128


def _edge_compute(HS, DQ, D2, ea, consts):
    full = lambda s: pl.BlockSpec(s, lambda i: (0,) * len(s))
    in_specs = [pl.BlockSpec((BE, D), lambda i: (i, 0)),
                pl.BlockSpec((BE, 2 * D), lambda i: (i, 0)),
                pl.BlockSpec((1, 1, BE), lambda i: (i, 0, 0)),
                pl.BlockSpec((BE, EF), lambda i: (i, 0))]
    in_specs += [full(c.shape) for c in consts]
    return pl.pallas_call(
        _edge_body,
        grid=(GE,),
        in_specs=in_specs,
        out_specs=[pl.BlockSpec((BE, D), lambda i: (i, 0)),
                   pl.BlockSpec((BE, D), lambda i: (i, 0))],
        out_shape=[jax.ShapeDtypeStruct((EC, D), jnp.float32),
                   jax.ShapeDtypeStruct((EC, D), jnp.float32)],
    )(HS, DQ, D2, ea, *consts)


_ROWS = 624
_TAIL0 = NS * _ROWS
_TAIL = N - _TAIL0


def _scatter_body(dst, pmx, z, o, idxv0, pmv0, idxv1, pmv1, tbl,
                  seml0, seml1, semx0, semx1):
    c = lax.axis_index("c")
    s = lax.axis_index("s")
    wid = s * NC + c
    r0 = s * _ROWS
    pltpu.sync_copy(z.at[pl.ds(r0, _ROWS)], tbl.at[pl.ds(r0, _ROWS)])

    @pl.when(s == 0)
    def _():
        pltpu.sync_copy(z.at[pl.ds(_TAIL0, _TAIL)], tbl.at[pl.ds(_TAIL0, _TAIL)])

    plsc.subcore_barrier()
    bufs = ((idxv0, pmv0, seml0, semx0), (idxv1, pmv1, seml1, semx1))

    def off_of(i):
        return pl.multiple_of((wid + i * NW) * CH, 8)

    def load(i, P):
        idxv, pmv, seml, _ = bufs[P]
        pltpu.async_copy(dst.at[pl.ds(off_of(i), CH)], idxv, seml)
        pltpu.async_copy(pmx.at[pl.ds(off_of(i), CH)], pmv, seml)

    def process(i, P, load_next, drain_other):
        idxv, pmv, seml, semx = bufs[P]
        pltpu.make_async_copy(dst.at[pl.ds(0, CH)], idxv, seml).wait()
        pltpu.make_async_copy(pmx.at[pl.ds(0, CH)], pmv, seml).wait()
        if drain_other is not None:
            oidxv, opmv, _, osemx = bufs[1 - P]

            @pl.when(drain_other)
            def _():
                pltpu.make_async_copy(opmv, tbl.at[oidxv], osemx).wait()
        if load_next:
            load(i + 1, 1 - P)
        pltpu.async_copy(pmv, tbl.at[idxv], semx, add=True)

    load(0, 0)

    def body(j, _):
        process(2 * j, 0, True, j > 0)
        process(2 * j + 1, 1, True, True)
        return 0

    lax.fori_loop(0, (ZT - 1) // 2, body, 0)
    process(ZT - 1, 0, False, True)
    pltpu.make_async_copy(pmv0, tbl.at[idxv0], semx0).wait()
    plsc.subcore_barrier()
    pltpu.sync_copy(tbl.at[pl.ds(r0, _ROWS)], o.at[c, pl.ds(r0, _ROWS)])

    @pl.when(s == 0)
    def _():
        pltpu.sync_copy(tbl.at[pl.ds(_TAIL0, _TAIL)], o.at[c, pl.ds(_TAIL0, _TAIL)])


@functools.cache
def _scatter():
    return pl.kernel(
        _scatter_body,
        out_type=jax.ShapeDtypeStruct((NC, N, D), jnp.float32),
        mesh=plsc.VectorSubcoreMesh(core_axis_name="c", subcore_axis_name="s",
                                    num_cores=NC, num_subcores=NS),
        scratch_types=2 * [pltpu.VMEM((CH,), jnp.int32),
                           pltpu.VMEM((CH, D), jnp.float32)]
                      + [pltpu.VMEM_SHARED((N, D), jnp.float32)]
                      + 4 * [pltpu.SemaphoreType.DMA],
    )


def _final_body(*refs):
    (m0, m1, m2, m3, m4, p0, p1, p2, p3, p4, h_ref,
     w1a, w1b, b1, g, be, w2, b2b, ones, o_ref) = refs
    s2 = sum(t[0] + t[1] for t in (m0, m1, m2, m3, m4))
    den = sum(t[0] + t[1] for t in (p0, p1, p2, p3, p4))
    agg = s2 / (den + 1e-16)
    hb = h_ref[...]
    hh = (jnp.dot(agg, w1a[...], preferred_element_type=jnp.float32)
          + jnp.dot(hb, w1b[...], preferred_element_type=jnp.float32) + b1[...])
    a = _ln_relu(hh, g[...], be[...], ones[...])
    o_ref[...] = jnp.dot(a, w2[...], preferred_element_type=jnp.float32) + b2b[...] + hb


def _final(parts_mv, parts_p, h, p):
    full = lambda s: pl.BlockSpec(s, lambda i: (0,) * len(s))
    part_spec = pl.BlockSpec((NC, BN, D), lambda i: (0, i, 0))
    return pl.pallas_call(
        _final_body,
        grid=(GN,),
        in_specs=2 * K * [part_spec]
                 + [pl.BlockSpec((BN, D), lambda i: (i, 0)),
                    full((D, D)), full((D, D)), full((1, D)),
                    full((1, D)), full((1, D)), full((D, D)), full((1, D)),
                    full((D, 1))],
        out_specs=pl.BlockSpec((BN, D), lambda i: (i, 0)),
        out_shape=jax.ShapeDtypeStruct((N, D), jnp.float32),
    )(*parts_mv, *parts_p, h, p['W1'][:D], p['W1'][D:],
      p['b1'][None], p['g'][None], p['be'][None], p['W2'], p['b2'][None],
      jnp.ones((D, 1), jnp.float32))


def kernel(x, h, edge_attr, edge_index, e_w, hk, hv, hq, ew_W, ew_b, nout):
    del e_w
    src = edge_index[0]
    dst = edge_index[1]

    q = _q_mlp(h, hq)
    tq = jnp.concatenate([h, q], axis=1)

    def mk_w1(p):
        w = p['W1']
        return jnp.concatenate(
            [w[0:84], jnp.zeros((44, D), jnp.float32), w[84:212], w[212:340]], axis=0)

    ewc = jnp.zeros((1, 128), jnp.float32).at[0, 4:84].set(ew_W[:, 0])
    consts = (jnp.asarray(_E1), jnp.asarray(_R1), jnp.asarray(_R2),
              jnp.asarray(_OFFC), ewc, ew_b[None],
              mk_w1(hk), hk['b1'][None], hk['g'][None], hk['be'][None],
              hk['W2'], hk['b2'][None],
              mk_w1(hv), hv['b1'][None], hv['g'][None], hv['be'][None],
              hv['W2'], hv['b2'][None],
              jnp.asarray(_S), jnp.asarray(_B2), jnp.ones((D, 1), jnp.float32))
    zn = jnp.zeros((N, D), jnp.float32)
    parts_mv, parts_p = [], []
    for k in range(K):
        sl = slice(k * EC, (k + 1) * EC)
        srck, dstk = src[sl], dst[sl]
        HSk, DQk, D2k = _gather()(h, tq, x[:, 0], x[:, 1], x[:, 2], srck, dstk)
        mvk, ppk = _edge_compute(HSk, DQk, D2k.reshape(GE, 1, BE),
                                 edge_attr[sl], consts)
        parts_mv.append(_scatter()(dstk, mvk, zn))
        parts_p.append(_scatter()(dstk, ppk, zn))
    return _final(parts_mv, parts_p, h, nout)
